# Initial kernel scaffold; baseline (speedup 1.0000x reference)
import jax
import jax.numpy as jnp
from jax.experimental import pallas as pl


def kernel(x, edge_index, W1_self, W1_neigh, b1, W2_self, W2_neigh, b2):
    raise NotImplementedError("write your pallas kernel here")


# speedup vs baseline: 3.7197x; 3.7197x over previous
import functools

import jax
import jax.numpy as jnp
from jax import lax
from jax.experimental import pallas as pl
from jax.experimental.pallas import tpu as pltpu
from jax.experimental.pallas import tpu_sc as plsc

N = 10000
E = 320000
D = 128

NC = 2
NS = 16
NW = NC * NS

CHUNK = 128
ROWS_PER_TILE = 640
N_PAD = NS * ROWS_PER_TILE

E_PAD = ((E + NW * CHUNK - 1) // (NW * CHUNK)) * (NW * CHUNK)
E_PER_TILE = E_PAD // NW
N_CHUNKS = E_PER_TILE // CHUNK
L = 16

_MESH = plsc.VectorSubcoreMesh(
    core_axis_name="c", subcore_axis_name="s", num_cores=NC, num_subcores=NS)


def _zero_acc(zeros_hbm, rows_v, acc_sh, s):
  base_r = s * ROWS_PER_TILE
  pltpu.sync_copy(zeros_hbm.at[pl.ds(0, CHUNK)], rows_v)
  for j in range(ROWS_PER_TILE // CHUNK):
    pltpu.sync_copy(rows_v, acc_sh.at[pl.ds(base_r + j * CHUNK, CHUNK)])


def _write_acc(acc_sh, rows_v, acc_out, c, s):
  base_r = s * ROWS_PER_TILE
  for j in range(ROWS_PER_TILE // CHUNK):
    r0 = base_r + j * CHUNK
    pltpu.sync_copy(acc_sh.at[pl.ds(r0, CHUNK)], rows_v)
    pltpu.sync_copy(rows_v, acc_out.at[c, pl.ds(r0, CHUNK)])


def _agg_body(x_hbm, src_hbm, dst_hbm, zeros_hbm, acc_out,
              src_v, dst_v, rows_v, acc_sh, sem):
  c = lax.axis_index("c")
  s = lax.axis_index("s")
  _zero_acc(zeros_hbm, rows_v, acc_sh, s)
  plsc.subcore_barrier()

  ebase = (c * NS + s) * E_PER_TILE

  def chunk_body(i, carry):
    off = ebase + i * CHUNK
    pltpu.sync_copy(src_hbm.at[pl.ds(off, CHUNK)], src_v)
    pltpu.sync_copy(dst_hbm.at[pl.ds(off, CHUNK)], dst_v)
    pltpu.async_copy(x_hbm.at[src_v], rows_v, sem).wait()
    pltpu.sync_copy(rows_v, acc_sh.at[dst_v], add=True)
    return carry

  lax.fori_loop(0, N_CHUNKS, chunk_body, 0)
  plsc.subcore_barrier()
  _write_acc(acc_sh, rows_v, acc_out, c, s)


_sc_agg = pl.kernel(
    _agg_body,
    out_type=jax.ShapeDtypeStruct((NC, N_PAD, D), jnp.float32),
    mesh=_MESH,
    scratch_types=[
        pltpu.VMEM((CHUNK,), jnp.int32),
        pltpu.VMEM((CHUNK,), jnp.int32),
        pltpu.VMEM((CHUNK, D), jnp.float32),
        pltpu.VMEM_SHARED((N_PAD, D), jnp.float32),
        pltpu.SemaphoreType.DMA,
    ])


def _deg_body(dst_hbm, zeros_hbm, deg_out, dst_v, ones_v, rows_v, deg_sh, sem):
  c = lax.axis_index("c")
  s = lax.axis_index("s")
  _zero_acc(zeros_hbm, rows_v, deg_sh, s)
  one = jnp.ones((L,), jnp.float32)
  def fill_ones(i, carry):
    ones_v[i // (D // L), pl.ds((i % (D // L)) * L, L)] = one
    return carry
  lax.fori_loop(0, CHUNK * (D // L), fill_ones, 0)
  plsc.subcore_barrier()

  ebase = (c * NS + s) * E_PER_TILE

  def chunk_body(i, carry):
    off = ebase + i * CHUNK
    pltpu.sync_copy(dst_hbm.at[pl.ds(off, CHUNK)], dst_v)
    pltpu.sync_copy(ones_v, deg_sh.at[dst_v], add=True)
    return carry

  lax.fori_loop(0, N_CHUNKS, chunk_body, 0)
  plsc.subcore_barrier()
  _write_acc(deg_sh, rows_v, deg_out, c, s)


_sc_deg = pl.kernel(
    _deg_body,
    out_type=jax.ShapeDtypeStruct((NC, N_PAD, D), jnp.float32),
    mesh=_MESH,
    scratch_types=[
        pltpu.VMEM((CHUNK,), jnp.int32),
        pltpu.VMEM((CHUNK, D), jnp.float32),
        pltpu.VMEM((CHUNK, D), jnp.float32),
        pltpu.VMEM_SHARED((N_PAD, D), jnp.float32),
        pltpu.SemaphoreType.DMA,
    ])


BLK = 1000


def _tc_layer_body(activation, x_ref, p0_ref, p1_ref, deg_ref,
                   ws_ref, wn_ref, b_ref, o_ref):
  inv = 1.0 / jnp.maximum(deg_ref[...], 1.0)
  mean = (p0_ref[...] + p1_ref[...]) * inv
  h = (jnp.dot(x_ref[...], ws_ref[...], preferred_element_type=jnp.float32)
       + jnp.dot(mean, wn_ref[...], preferred_element_type=jnp.float32)
       + b_ref[...])
  if activation == "relu":
    o_ref[...] = jnp.maximum(h, 0.0)
  else:
    m = jnp.max(h, axis=1, keepdims=True)
    z = h - m
    lse = jnp.log(jnp.sum(jnp.exp(z), axis=1, keepdims=True))
    o_ref[...] = z - lse


def _tc_layer(x, p0, p1, deg, w_self, w_neigh, b, activation):
  grid = (N // BLK,)
  row_spec = pl.BlockSpec((BLK, D), lambda i: (i, 0))
  deg_spec = pl.BlockSpec((BLK, 1), lambda i: (i, 0))
  full_spec = pl.BlockSpec((D, D), lambda i: (0, 0))
  b_spec = pl.BlockSpec((1, D), lambda i: (0, 0))
  return pl.pallas_call(
      functools.partial(_tc_layer_body, activation),
      grid=grid,
      in_specs=[row_spec, row_spec, row_spec, deg_spec,
                full_spec, full_spec, b_spec],
      out_specs=row_spec,
      out_shape=jax.ShapeDtypeStruct((N, D), jnp.float32),
  )(x, p0, p1, deg, w_self, w_neigh, b.reshape(1, D))


@jax.jit
def kernel(x, edge_index, W1_self, W1_neigh, b1, W2_self, W2_neigh, b2):
  src = edge_index[0]
  dst = edge_index[1]
  pad = E_PAD - E
  src_p = jnp.concatenate([src, jnp.zeros((pad,), jnp.int32)])
  dst_p = jnp.concatenate([dst, jnp.full((pad,), N, jnp.int32)])

  zeros = jnp.zeros((CHUNK, D), jnp.float32)

  deg_p = _sc_deg(dst_p, zeros)
  deg = (deg_p[0, :N, 0] + deg_p[1, :N, 0]).reshape(N, 1)
  agg1 = _sc_agg(x, src_p, dst_p, zeros)
  h = _tc_layer(x, agg1[0, :N], agg1[1, :N], deg,
                W1_self, W1_neigh, b1, "relu")
  agg2 = _sc_agg(h, src_p, dst_p, zeros)
  out = _tc_layer(h, agg2[0, :N], agg2[1, :N], deg,
                  W2_self, W2_neigh, b2, "log_softmax")
  return out
